# baseline (speedup 1.0000x reference)
import functools
import math

import jax
import jax.numpy as jnp
import numpy as np
from jax import lax
from jax.experimental import pallas as pl
from jax.experimental.pallas import tpu as pltpu
from jax.experimental.pallas import tpu_sc as plsc

D_MODEL = 128
CONTEXT = 2048
NC, NS, L = 2, 16, 16
NW = NC * NS
NBUF = 3
CHB = 2


def _make_pe2(context_size, d_model):
    position = np.arange(context_size, dtype=np.float32)[:, None]
    div_term = np.exp(
        np.arange(0, d_model, 2, dtype=np.float32) * (-math.log(10000.0) / d_model)
    )
    pe = np.zeros((context_size, d_model), dtype=np.float32)
    pe[:, 0::2] = np.sin(position * div_term)
    pe[:, 1::2] = np.cos(position * div_term)
    return jnp.asarray(2.0 * pe)


def _embed_body(xr_hbm, table_hbm, pe2_hbm, out_hbm,
                idx_v, pe_v, gbuf, obuf, gsems, wsems):
    _, n_idx = xr_hbm.shape
    b_total, s_len, _ = out_hbm.shape
    sl_len = s_len // NW
    rows = CHB * sl_len
    n_chunks = b_total // CHB
    n_laps = (n_chunks + NBUF - 1) // NBUF
    w = lax.axis_index("s") * NC + lax.axis_index("c")
    base = w * sl_len

    pltpu.sync_copy(pe2_hbm.at[pl.ds(base, sl_len)], pe_v)
    pltpu.sync_copy(xr_hbm.at[w], idx_v)

    def gather(c, k):
        pltpu.async_copy(
            table_hbm.at[idx_v.at[pl.ds(c * rows, rows)]], gbuf.at[k],
            gsems.at[k],
        )

    def wb_copy(c, k, i):
        return pltpu.make_async_copy(
            obuf.at[k, pl.ds(i * sl_len, sl_len)],
            out_hbm.at[c * CHB + i, pl.ds(base, sl_len)],
            wsems.at[k],
        )

    for k in range(NBUF):
        if k < 32 // CHB:
            gather(k, k)

    def outer(g, carry):
        for k in range(NBUF):
            c = g * NBUF + k

            @pl.when(c < n_chunks)
            def _():
                pltpu.make_async_copy(
                    table_hbm.at[idx_v.at[pl.ds(c * rows, rows)]], gbuf.at[k],
                    gsems.at[k],
                ).wait()

                @pl.when(c >= NBUF)
                def _():
                    for i in range(CHB):
                        wb_copy(c, k, i).wait()

                @plsc.parallel_loop(0, rows, unroll=4)
                def _row(r):
                    for j in range(D_MODEL // L):
                        sl = pl.ds(j * L, L)
                        obuf[k, r, sl] = gbuf[k, r, sl] * 2.0 + pe_v[r % sl_len, sl]

                for i in range(CHB):
                    wb_copy(c, k, i).start()
                nc = c + NBUF

                @pl.when(nc < n_chunks)
                def _():
                    gather(nc, k)

        return carry

    lax.fori_loop(0, n_laps, outer, 0)

    for dk in range(NBUF):
        c = n_chunks - NBUF + dk
        for i in range(CHB):
            wb_copy(c, c % NBUF, i).wait()


@jax.jit
def kernel(x, table):
    b, s_len = x.shape
    pe2 = _make_pe2(CONTEXT, D_MODEL)[:s_len]
    sl_len = s_len // NW
    rows = CHB * sl_len
    mesh = plsc.VectorSubcoreMesh(
        core_axis_name="c", subcore_axis_name="s", num_cores=NC, num_subcores=NS
    )
    xr = (
        x.astype(jnp.int32)
        .reshape(b, NW, sl_len)
        .transpose(1, 0, 2)
        .reshape(NW, b * sl_len)
    )
    run = functools.partial(
        pl.kernel,
        out_type=jax.ShapeDtypeStruct((b, s_len, D_MODEL), jnp.float32),
        mesh=mesh,
        scratch_types=[
            pltpu.VMEM((b * sl_len,), jnp.int32),
            pltpu.VMEM((sl_len, D_MODEL), jnp.float32),
            pltpu.VMEM((NBUF, rows, D_MODEL), jnp.float32),
            pltpu.VMEM((NBUF, rows, D_MODEL), jnp.float32),
            pltpu.SemaphoreType.DMA((NBUF,)),
            pltpu.SemaphoreType.DMA((NBUF,)),
        ],
    )(_embed_body)
    return run(xr, table, pe2)


# speedup vs baseline: 1.0116x; 1.0116x over previous
import functools
import math

import jax
import jax.numpy as jnp
import numpy as np
from jax import lax
from jax.experimental import pallas as pl
from jax.experimental.pallas import tpu as pltpu
from jax.experimental.pallas import tpu_sc as plsc

D_MODEL = 128
CONTEXT = 2048
NC, NS, L = 2, 16, 16
NW = NC * NS
NBUF = 3
CHB = 2


def _make_pe2(context_size, d_model):
    position = np.arange(context_size, dtype=np.float32)[:, None]
    div_term = np.exp(
        np.arange(0, d_model, 2, dtype=np.float32) * (-math.log(10000.0) / d_model)
    )
    pe = np.zeros((context_size, d_model), dtype=np.float32)
    pe[:, 0::2] = np.sin(position * div_term)
    pe[:, 1::2] = np.cos(position * div_term)
    return jnp.asarray(2.0 * pe)


def _embed_body(xf_hbm, table_hbm, pe2_hbm, out_hbm,
                idx_v, pe_v, gbuf, obuf, gsems, wsems, isem):
    b_total, s_len, _ = out_hbm.shape
    sl_len = s_len // NW
    rows = CHB * sl_len
    n_chunks = b_total // CHB
    n_laps = (n_chunks + NBUF - 1) // NBUF
    w = lax.axis_index("s") * NC + lax.axis_index("c")
    base = pl.multiple_of(w * sl_len, sl_len)

    for bb in range(b_total):
        pltpu.async_copy(
            xf_hbm.at[pl.ds(bb * s_len + base, sl_len)],
            idx_v.at[pl.ds(bb * sl_len, sl_len)],
            isem,
        )
    pltpu.sync_copy(pe2_hbm.at[pl.ds(base, sl_len)], pe_v)
    for bb in range(b_total):
        pltpu.make_async_copy(
            xf_hbm.at[pl.ds(bb * s_len + base, sl_len)],
            idx_v.at[pl.ds(bb * sl_len, sl_len)],
            isem,
        ).wait()

    def gather(c, k):
        pltpu.async_copy(
            table_hbm.at[idx_v.at[pl.ds(c * rows, rows)]], gbuf.at[k],
            gsems.at[k],
        )

    def wb_copy(c, k, i):
        return pltpu.make_async_copy(
            obuf.at[k, pl.ds(i * sl_len, sl_len)],
            out_hbm.at[c * CHB + i, pl.ds(base, sl_len)],
            wsems.at[k],
        )

    for k in range(NBUF):
        if k < 32 // CHB:
            gather(k, k)

    def outer(g, carry):
        for k in range(NBUF):
            c = g * NBUF + k

            @pl.when(c < n_chunks)
            def _():
                pltpu.make_async_copy(
                    table_hbm.at[idx_v.at[pl.ds(c * rows, rows)]], gbuf.at[k],
                    gsems.at[k],
                ).wait()

                @pl.when(c >= NBUF)
                def _():
                    for i in range(CHB):
                        wb_copy(c, k, i).wait()

                @plsc.parallel_loop(0, rows, unroll=4)
                def _row(r):
                    for j in range(D_MODEL // L):
                        sl = pl.ds(j * L, L)
                        obuf[k, r, sl] = gbuf[k, r, sl] * 2.0 + pe_v[r % sl_len, sl]

                for i in range(CHB):
                    wb_copy(c, k, i).start()
                nc = c + NBUF

                @pl.when(nc < n_chunks)
                def _():
                    gather(nc, k)

        return carry

    lax.fori_loop(0, n_laps, outer, 0)

    for dk in range(NBUF):
        c = n_chunks - NBUF + dk
        for i in range(CHB):
            wb_copy(c, c % NBUF, i).wait()


@jax.jit
def kernel(x, table):
    b, s_len = x.shape
    pe2 = _make_pe2(CONTEXT, D_MODEL)[:s_len]
    sl_len = s_len // NW
    rows = CHB * sl_len
    mesh = plsc.VectorSubcoreMesh(
        core_axis_name="c", subcore_axis_name="s", num_cores=NC, num_subcores=NS
    )
    xf = x.astype(jnp.int32).reshape(b * s_len)
    run = functools.partial(
        pl.kernel,
        out_type=jax.ShapeDtypeStruct((b, s_len, D_MODEL), jnp.float32),
        mesh=mesh,
        scratch_types=[
            pltpu.VMEM((b * sl_len,), jnp.int32),
            pltpu.VMEM((sl_len, D_MODEL), jnp.float32),
            pltpu.VMEM((NBUF, rows, D_MODEL), jnp.float32),
            pltpu.VMEM((NBUF, rows, D_MODEL), jnp.float32),
            pltpu.SemaphoreType.DMA((NBUF,)),
            pltpu.SemaphoreType.DMA((NBUF,)),
            pltpu.SemaphoreType.DMA,
        ],
    )(_embed_body)
    return run(xf, table, pe2)


# speedup vs baseline: 1.0301x; 1.0184x over previous
import functools
import math

import jax
import jax.numpy as jnp
import numpy as np
from jax import lax
from jax.experimental import pallas as pl
from jax.experimental.pallas import tpu as pltpu
from jax.experimental.pallas import tpu_sc as plsc

D_MODEL = 128
CONTEXT = 2048
NC, NS, L = 2, 16, 16
NW = NC * NS
NBUF = 4
ALIGN = 128


def _make_pe2(context_size, d_model):
    position = np.arange(context_size, dtype=np.float32)[:, None]
    div_term = np.exp(
        np.arange(0, d_model, 2, dtype=np.float32) * (-math.log(10000.0) / d_model)
    )
    pe = np.zeros((context_size, d_model), dtype=np.float32)
    pe[:, 0::2] = np.sin(position * div_term)
    pe[:, 1::2] = np.cos(position * div_term)
    return jnp.asarray(2.0 * pe)


def _embed_body(x_hbm, table_hbm, pe2_hbm, out_hbm,
                idx_v, pe_v, gbuf, obuf, gsems, wsems, isem):
    b_total, s_len = x_hbm.shape
    sl_len = s_len // NW
    w = lax.axis_index("s") * NC + lax.axis_index("c")
    base = pl.multiple_of(w * sl_len, sl_len)
    aligned = pl.multiple_of((w // (ALIGN // sl_len)) * ALIGN, ALIGN)
    sub = base - aligned

    def stage_idx(bb):
        return pltpu.make_async_copy(
            x_hbm.at[bb, pl.ds(aligned, ALIGN)], idx_v.at[bb], isem
        )

    def gather(b, k):
        pltpu.async_copy(
            table_hbm.at[idx_v.at[b, pl.ds(sub, sl_len)]], gbuf.at[k],
            gsems.at[k],
        )

    for bb in range(NBUF):
        stage_idx(bb).start()
    for bb in range(NBUF):
        stage_idx(bb).wait()
    for k in range(NBUF):
        gather(k, k)
    for bb in range(NBUF, b_total):
        stage_idx(bb).start()
    pltpu.sync_copy(pe2_hbm.at[pl.ds(base, sl_len)], pe_v)
    for bb in range(NBUF, b_total):
        stage_idx(bb).wait()

    def outer(g, carry):
        for k in range(NBUF):
            b = g * NBUF + k
            pltpu.make_async_copy(
                table_hbm.at[idx_v.at[b, pl.ds(sub, sl_len)]], gbuf.at[k],
                gsems.at[k],
            ).wait()

            @pl.when(b >= NBUF)
            def _():
                pltpu.make_async_copy(
                    obuf.at[k], out_hbm.at[b, pl.ds(base, sl_len)], wsems.at[k]
                ).wait()

            @plsc.parallel_loop(0, sl_len, unroll=4)
            def _row(r):
                for j in range(D_MODEL // L):
                    sl = pl.ds(j * L, L)
                    obuf[k, r, sl] = gbuf[k, r, sl] * 2.0 + pe_v[r, sl]

            pltpu.async_copy(
                obuf.at[k], out_hbm.at[b, pl.ds(base, sl_len)], wsems.at[k]
            )
            nb = b + NBUF

            @pl.when(nb < b_total)
            def _():
                gather(nb, k)

        return carry

    lax.fori_loop(0, b_total // NBUF, outer, 0)

    for k in range(NBUF):
        b = b_total - NBUF + k
        pltpu.make_async_copy(
            obuf.at[k], out_hbm.at[b, pl.ds(base, sl_len)], wsems.at[k]
        ).wait()


@jax.jit
def kernel(x, table):
    b, s_len = x.shape
    pe2 = _make_pe2(CONTEXT, D_MODEL)[:s_len]
    sl_len = s_len // NW
    mesh = plsc.VectorSubcoreMesh(
        core_axis_name="c", subcore_axis_name="s", num_cores=NC, num_subcores=NS
    )
    run = functools.partial(
        pl.kernel,
        out_type=jax.ShapeDtypeStruct((b, s_len, D_MODEL), jnp.float32),
        mesh=mesh,
        scratch_types=[
            pltpu.VMEM((b, ALIGN), jnp.int32),
            pltpu.VMEM((sl_len, D_MODEL), jnp.float32),
            pltpu.VMEM((NBUF, sl_len, D_MODEL), jnp.float32),
            pltpu.VMEM((NBUF, sl_len, D_MODEL), jnp.float32),
            pltpu.SemaphoreType.DMA((NBUF,)),
            pltpu.SemaphoreType.DMA((NBUF,)),
            pltpu.SemaphoreType.DMA,
        ],
    )(_embed_body)
    return run(x.astype(jnp.int32), table, pe2)


# speedup vs baseline: 1.0428x; 1.0123x over previous
import functools
import math

import jax
import jax.numpy as jnp
import numpy as np
from jax import lax
from jax.experimental import pallas as pl
from jax.experimental.pallas import tpu as pltpu
from jax.experimental.pallas import tpu_sc as plsc

D_MODEL = 128
CONTEXT = 2048
NC, NS, L = 2, 16, 16
NW = NC * NS
NBUF = 4
ALIGN = 128


def _make_pe2(context_size, d_model):
    position = np.arange(context_size, dtype=np.float32)[:, None]
    div_term = np.exp(
        np.arange(0, d_model, 2, dtype=np.float32) * (-math.log(10000.0) / d_model)
    )
    pe = np.zeros((context_size, d_model), dtype=np.float32)
    pe[:, 0::2] = np.sin(position * div_term)
    pe[:, 1::2] = np.cos(position * div_term)
    return jnp.asarray(2.0 * pe)


def _embed_body(x_hbm, table_hbm, pe2_hbm, out_hbm,
                idx_v, pe_v, gbuf, obuf, gsems, wsems, isem):
    b_total, s_len = x_hbm.shape
    sl_len = s_len // NW
    w = lax.axis_index("s") * NC + lax.axis_index("c")
    base = pl.multiple_of(w * sl_len, sl_len)
    aligned = pl.multiple_of((w // (ALIGN // sl_len)) * ALIGN, ALIGN)
    sub = base - aligned

    def stage_idx(bb):
        return pltpu.make_async_copy(
            x_hbm.at[bb, pl.ds(aligned, ALIGN)], idx_v.at[bb], isem
        )

    def gather(b, k):
        pltpu.async_copy(
            table_hbm.at[idx_v.at[b, pl.ds(sub, sl_len)]], gbuf.at[k],
            gsems.at[k],
        )

    for bb in range(NBUF):
        stage_idx(bb).start()
    for bb in range(NBUF):
        stage_idx(bb).wait()
    for k in range(NBUF):
        gather(k, k)
    def stage_rest(bb, carry):
        stage_idx(bb).start()
        return carry

    lax.fori_loop(NBUF, b_total, stage_rest, 0)
    pltpu.sync_copy(pe2_hbm.at[pl.ds(base, sl_len)], pe_v)

    def drain_rest(bb, carry):
        stage_idx(bb).wait()
        return carry

    lax.fori_loop(NBUF, b_total, drain_rest, 0)

    def outer(g, carry):
        for k in range(NBUF):
            b = g * NBUF + k
            pltpu.make_async_copy(
                table_hbm.at[idx_v.at[b, pl.ds(sub, sl_len)]], gbuf.at[k],
                gsems.at[k],
            ).wait()

            @pl.when(b >= NBUF)
            def _():
                pltpu.make_async_copy(
                    obuf.at[k], out_hbm.at[b, pl.ds(base, sl_len)], wsems.at[k]
                ).wait()

            @plsc.parallel_loop(0, sl_len, unroll=2)
            def _row(r):
                for j in range(D_MODEL // L):
                    sl = pl.ds(j * L, L)
                    obuf[k, r, sl] = gbuf[k, r, sl] * 2.0 + pe_v[r, sl]

            pltpu.async_copy(
                obuf.at[k], out_hbm.at[b, pl.ds(base, sl_len)], wsems.at[k]
            )
            nb = b + NBUF

            @pl.when(nb < b_total)
            def _():
                gather(nb, k)

        return carry

    lax.fori_loop(0, b_total // NBUF, outer, 0)

    for k in range(NBUF):
        b = b_total - NBUF + k
        pltpu.make_async_copy(
            obuf.at[k], out_hbm.at[b, pl.ds(base, sl_len)], wsems.at[k]
        ).wait()


@jax.jit
def kernel(x, table):
    b, s_len = x.shape
    pe2 = _make_pe2(CONTEXT, D_MODEL)[:s_len]
    sl_len = s_len // NW
    mesh = plsc.VectorSubcoreMesh(
        core_axis_name="c", subcore_axis_name="s", num_cores=NC, num_subcores=NS
    )
    run = functools.partial(
        pl.kernel,
        out_type=jax.ShapeDtypeStruct((b, s_len, D_MODEL), jnp.float32),
        mesh=mesh,
        scratch_types=[
            pltpu.VMEM((b, ALIGN), jnp.int32),
            pltpu.VMEM((sl_len, D_MODEL), jnp.float32),
            pltpu.VMEM((NBUF, sl_len, D_MODEL), jnp.float32),
            pltpu.VMEM((NBUF, sl_len, D_MODEL), jnp.float32),
            pltpu.SemaphoreType.DMA((NBUF,)),
            pltpu.SemaphoreType.DMA((NBUF,)),
            pltpu.SemaphoreType.DMA,
        ],
    )(_embed_body)
    return run(x.astype(jnp.int32), table, pe2)
